# baseline (speedup 1.0000x reference)
import functools

import jax
import jax.numpy as jnp
from jax import lax
from jax.experimental import pallas as pl
from jax.experimental.pallas import tpu as pltpu
from jax.experimental.pallas import tpu_sc as plsc

_NC = 2
_NS = 16
_CH = 128


def _sc_segment_sums(tables, keys, distp, zeros_h2, zeros_de, ones_src,
                     NP, RT, EW):
    DH = tables.shape[2]
    DE = distp.shape[1]
    NB = 8
    nblocks = EW // (_CH * NB)
    f32 = jnp.float32
    mesh = plsc.VectorSubcoreMesh(core_axis_name="c", subcore_axis_name="s")
    out_type = [
        jax.ShapeDtypeStruct((_NC, NP, DH), f32),
        jax.ShapeDtypeStruct((_NC, NP, DE), f32),
        jax.ShapeDtypeStruct((_NC, NP, DE), f32),
    ]
    scratch_types = [
        pltpu.VMEM((NB, _CH), jnp.int32),
        pltpu.VMEM((NB, _CH), jnp.int32),
        pltpu.VMEM((NB, _CH), jnp.int32),
        pltpu.VMEM((NB * _CH, DE), f32),
        pltpu.VMEM((_CH, DH), f32),
        pltpu.VMEM((_CH, DH), f32),
        pltpu.VMEM((_CH, DE), f32),
        pltpu.VMEM_SHARED((NP, DH), f32),
        pltpu.VMEM_SHARED((NP, DE), f32),
        pltpu.VMEM_SHARED((NP, DE), f32),
        pltpu.SemaphoreType.DMA,
        pltpu.SemaphoreType.DMA,
        pltpu.SemaphoreType.DMA,
    ]

    @functools.partial(pl.kernel, mesh=mesh, out_type=out_type,
                       scratch_types=scratch_types,
                       compiler_params=pltpu.CompilerParams(
                           use_tc_tiling_on_sc=False))
    def sck(tables_h, keys_h, distp_h, zeros_h2_h, zeros_de_h, ones_h,
            o_nbr, o_ds, o_ct,
            self_b, nbr_b, key_b, dist_b, rows0, rows1, ones_v,
            a_nbr, a_ds, a_ct, sem0, sem1, sem_s):
        c = lax.axis_index("c")
        s = lax.axis_index("s")
        r0 = s * RT
        pltpu.sync_copy(zeros_h2_h, a_nbr.at[pl.ds(r0, RT), :])
        pltpu.sync_copy(zeros_de_h, a_ds.at[pl.ds(r0, RT), :])
        pltpu.sync_copy(zeros_de_h, a_ct.at[pl.ds(r0, RT), :])
        pltpu.sync_copy(ones_h, ones_v)
        plsc.subcore_barrier()

        chunk0 = s * (EW // _CH)
        rows = (rows0, rows1)
        sems = (sem0, sem1)

        def body(b, carry):
            crow = chunk0 + b * NB
            eoff = crow * _CH
            pltpu.sync_copy(keys_h.at[0, pl.ds(crow, NB), :], self_b)
            pltpu.sync_copy(keys_h.at[1, pl.ds(crow, NB), :], nbr_b)
            pltpu.sync_copy(keys_h.at[c, pl.ds(crow, NB), :], key_b)
            pltpu.sync_copy(distp_h.at[pl.ds(eoff, NB * _CH), :], dist_b)
            pend = pltpu.async_copy(
                tables_h.at[c].at[nbr_b.at[0]], rows[0], sems[0])
            for j in range(NB):
                nxt = None
                if j + 1 < NB:
                    nxt = pltpu.async_copy(
                        tables_h.at[c].at[nbr_b.at[j + 1]],
                        rows[(j + 1) % 2], sems[(j + 1) % 2])
                pend.wait()
                pend = nxt
                pltpu.sync_copy(rows[j % 2], a_nbr.at[self_b.at[j]],
                                add=True)
                pltpu.sync_copy(dist_b.at[pl.ds(j * _CH, _CH), :],
                                a_ds.at[key_b.at[j]], add=True)
                pltpu.sync_copy(ones_v, a_ct.at[key_b.at[j]], add=True)
            return carry

        lax.fori_loop(0, nblocks, body, 0)
        plsc.subcore_barrier()
        pltpu.sync_copy(a_nbr.at[pl.ds(r0, RT), :], o_nbr.at[c, pl.ds(r0, RT), :])
        pltpu.sync_copy(a_ds.at[pl.ds(r0, RT), :], o_ds.at[c, pl.ds(r0, RT), :])
        pltpu.sync_copy(a_ct.at[pl.ds(r0, RT), :], o_ct.at[c, pl.ds(r0, RT), :])

    return sck(tables, keys, distp, zeros_h2, zeros_de, ones_src)


def _gram_dist(dist):
    E, DE = dist.shape
    nsteps = 10
    BLK = -(-E // (nsteps * 8)) * 8
    M = BLK * nsteps
    if M != E:
        dist = jnp.pad(dist, ((0, M - E), (0, 0)))

    def gk(r_ref, o_ref):
        @pl.when(pl.program_id(0) == 0)
        def _init():
            o_ref[...] = jnp.zeros_like(o_ref)

        x = r_ref[...]
        o_ref[...] += lax.dot_general(
            x, x, (((0,), (0,)), ((), ())), preferred_element_type=jnp.float32)

    return pl.pallas_call(
        gk,
        grid=(nsteps,),
        in_specs=[pl.BlockSpec((BLK, DE), lambda i: (i, 0))],
        out_specs=pl.BlockSpec((DE, DE), lambda i: (0, 0)),
        out_shape=jax.ShapeDtypeStruct((DE, DE), jnp.float32),
    )(dist)


def _fused_dense(atom, nbrp, dsp, ctp, gdd_in,
                 w1a, w1b, b1, g1, be1, w2, b2, g2, be2, N, E):
    D = atom.shape[1]
    DH = D // 2
    DE = w1b.shape[1]
    Ef = float(E)

    def bk(atom_r, nbr_r, ds_r, ct_r, gdd_r,
           w1a0_r, w1a1_r, w1a_r, w1b_r, b1_r, g1_r, be1_r, w2_r, b2_r,
           g2_r, be2_r, o_r):
        nbr0 = nbr_r[0, :N, :]
        nbr1 = nbr_r[1, :N, :]
        dst = ds_r[0, :N, :]
        sv = ds_r[1, :N, :]
        cnt = ct_r[0, :N, 0:1]
        cb = ct_r[1, :N, 0:1]
        atom_v = atom_r[...]
        w1a0_v = w1a0_r[...]
        w1a1_v = w1a1_r[...]
        w1b_v = w1b_r[...]
        b1_v = b1_r[...]
        cc = jnp.maximum(cnt, 1.0)
        fea_pre = (
            lax.dot_general(nbr0 / cc, w1a0_v, (((1,), (1,)), ((), ())),
                            preferred_element_type=jnp.float32)
            + lax.dot_general(nbr1 / cc, w1a1_v, (((1,), (1,)), ((), ())),
                              preferred_element_type=jnp.float32)
            + lax.dot_general(dst / cc, w1b_v, (((1,), (1,)), ((), ())),
                              preferred_element_type=jnp.float32)
            + b1_v)
        g_nbr0 = jnp.sum(nbr0, axis=0, keepdims=True)
        g_nbr1 = jnp.sum(nbr1, axis=0, keepdims=True)
        g_dist = jnp.sum(dst, axis=0, keepdims=True)
        m1 = (
            lax.dot_general(g_nbr0, w1a0_v, (((1,), (1,)), ((), ())),
                            preferred_element_type=jnp.float32)
            + lax.dot_general(g_nbr1, w1a1_v, (((1,), (1,)), ((), ())),
                              preferred_element_type=jnp.float32)
            + lax.dot_general(g_dist, w1b_v, (((1,), (1,)), ((), ())),
                              preferred_element_type=jnp.float32)
        ) / Ef + b1_v
        gaa = lax.dot_general(atom_v * cb, atom_v, (((0,), (0,)), ((), ())),
                              preferred_element_type=jnp.float32)
        gad = lax.dot_general(atom_v, sv, (((0,), (0,)), ((), ())),
                              preferred_element_type=jnp.float32)
        gdd = gdd_r[...]
        w1a_v = w1a_r[...]
        t1 = lax.dot_general(w1a_v, gaa, (((1,), (0,)), ((), ())),
                             preferred_element_type=jnp.float32)
        t2 = lax.dot_general(w1a_v, gad, (((1,), (0,)), ((), ())),
                             preferred_element_type=jnp.float32)
        t3 = lax.dot_general(w1b_v, gdd, (((1,), (0,)), ((), ())),
                             preferred_element_type=jnp.float32)
        wgw = (jnp.sum(t1 * w1a_v, axis=1) + 2.0 * jnp.sum(t2 * w1b_v, axis=1)
               + jnp.sum(t3 * w1b_v, axis=1))
        m1f = m1[0, :]
        v1 = wgw / Ef + 2.0 * b1_v * m1f - b1_v * b1_v - m1f * m1f
        s1 = g1_r[...] / jnp.sqrt(v1 + 1e-5)
        mask = (cnt > 0.0).astype(jnp.float32)
        fea_summed = ((fea_pre - m1f) * s1 + be1_r[...]) * mask
        z = atom_v * mask
        h = lax.dot_general(z, w2_r[...], (((1,), (1,)), ((), ())),
                            preferred_element_type=jnp.float32) + b2_r[...]
        m2 = jnp.mean(h, axis=0)
        d2 = h - m2
        v2 = jnp.mean(d2 * d2, axis=0)
        xbn = d2 / jnp.sqrt(v2 + 1e-5) * g2_r[...] + be2_r[...] + fea_summed
        o_r[...] = jnp.maximum(xbn, 0.0) + jnp.log1p(jnp.exp(-jnp.abs(xbn)))

    return pl.pallas_call(
        bk,
        out_shape=jax.ShapeDtypeStruct((N, D), jnp.float32),
        compiler_params=pltpu.CompilerParams(
            vmem_limit_bytes=100 * 1024 * 1024),
    )(atom, nbrp, dsp, ctp, gdd_in,
      w1a[:, :DH], w1a[:, DH:], w1a, w1b, b1, g1, be1, w2, b2, g2, be2)


def kernel(atom_fea, nbr_dist_fea, nbr_adj_value, nbr_bond_type, self_fea_idx,
           nbr_fea_idx, ads_atom_idx, W1, b1, g1, be1, W2, b2, g2, be2):
    N, D = atom_fea.shape
    E, DE = nbr_dist_fea.shape
    DH = D // 2
    EW = -(-E // (_NS * _CH * 8)) * _CH * 8
    EP = EW * _NS
    RT = -(-(N + 1) // (_NS * 8)) * 8
    NP = RT * _NS

    f32 = jnp.float32
    selfp = jnp.pad(self_fea_idx.astype(jnp.int32), (0, EP - E),
                    constant_values=N)
    nbrp = jnp.pad(nbr_fea_idx.astype(jnp.int32), (0, EP - E),
                   constant_values=N)
    keys = jnp.stack([selfp, nbrp]).reshape(2, EP // _CH, _CH)
    distp = jnp.pad(nbr_dist_fea.astype(f32), ((0, EP - E), (0, 0)))
    table = jnp.pad(atom_fea.astype(f32), ((0, NP - N), (0, 0)))
    tables = jnp.stack([table[:, :DH], table[:, DH:]])
    zeros_h2 = jnp.zeros((RT, DH), f32)
    zeros_de = jnp.zeros((RT, DE), f32)
    ones_src = jnp.ones((_CH, DE), f32)

    nbr_part, ds_part, ct_part = _sc_segment_sums(
        tables, keys, distp, zeros_h2, zeros_de, ones_src, NP, RT, EW)

    gdd = _gram_dist(nbr_dist_fea.astype(f32))

    out = _fused_dense(
        atom_fea.astype(f32), nbr_part, ds_part, ct_part,
        gdd, W1[:, :D], W1[:, D:], b1, g1, be1, W2, b2, g2, be2, N, E)
    return out


# speedup vs baseline: 1.1455x; 1.1455x over previous
import functools

import jax
import jax.numpy as jnp
from jax import lax
from jax.experimental import pallas as pl
from jax.experimental.pallas import tpu as pltpu
from jax.experimental.pallas import tpu_sc as plsc

_NC = 2
_NS = 16
_CH = 128


def _sc_segment_sums(tables, keys, distp, zeros_h2, zeros_de, ones_src,
                     NP, RT, EW):
    DH = tables.shape[2]
    DE = distp.shape[1]
    NB = 8
    nblocks = EW // (_CH * NB)
    f32 = jnp.float32
    mesh = plsc.VectorSubcoreMesh(core_axis_name="c", subcore_axis_name="s")
    out_type = [
        jax.ShapeDtypeStruct((_NC, NP, DH), f32),
        jax.ShapeDtypeStruct((_NC, NP, DE), f32),
        jax.ShapeDtypeStruct((_NC, NP, DE), f32),
    ]
    scratch_types = [
        pltpu.VMEM((NB, _CH), jnp.int32),
        pltpu.VMEM((NB, _CH), jnp.int32),
        pltpu.VMEM((NB, _CH), jnp.int32),
        pltpu.VMEM((NB * _CH, DE), f32),
        pltpu.VMEM((_CH, DH), f32),
        pltpu.VMEM((_CH, DH), f32),
        pltpu.VMEM((_CH, DE), f32),
        pltpu.VMEM_SHARED((NP, DH), f32),
        pltpu.VMEM_SHARED((NP, DE), f32),
        pltpu.VMEM_SHARED((NP, DE), f32),
        pltpu.SemaphoreType.DMA,
        pltpu.SemaphoreType.DMA,
        pltpu.SemaphoreType.DMA,
        pltpu.SemaphoreType.DMA,
    ]

    @functools.partial(pl.kernel, mesh=mesh, out_type=out_type,
                       scratch_types=scratch_types,
                       compiler_params=pltpu.CompilerParams(
                           use_tc_tiling_on_sc=False))
    def sck(tables_h, keys_h, distp_h, zeros_h2_h, zeros_de_h, ones_h,
            o_nbr, o_ds, o_ct,
            self_b, nbr_b, key_b, dist_b, rows0, rows1, ones_v,
            a_nbr, a_ds, a_ct, sem0, sem1, sem_r, sem_s):
        c = lax.axis_index("c")
        s = lax.axis_index("s")
        r0 = s * RT
        pltpu.sync_copy(zeros_h2_h, a_nbr.at[pl.ds(r0, RT), :])
        pltpu.sync_copy(zeros_de_h, a_ds.at[pl.ds(r0, RT), :])
        pltpu.sync_copy(zeros_de_h, a_ct.at[pl.ds(r0, RT), :])
        pltpu.sync_copy(ones_h, ones_v)
        plsc.subcore_barrier()

        chunk0 = s * (EW // _CH)
        rows = (rows0, rows1)
        sems = (sem0, sem1)

        def body(b, carry):
            crow = chunk0 + b * NB
            eoff = crow * _CH
            pltpu.sync_copy(keys_h.at[0, pl.ds(crow, NB), :], self_b)
            pltpu.sync_copy(keys_h.at[1, pl.ds(crow, NB), :], nbr_b)
            pltpu.sync_copy(keys_h.at[c, pl.ds(crow, NB), :], key_b)
            pltpu.sync_copy(distp_h.at[pl.ds(eoff, NB * _CH), :], dist_b)
            pend = pltpu.async_copy(
                tables_h.at[c].at[nbr_b.at[0]], rows[0], sems[0])
            row_sc = None
            small_sc = []
            for j in range(NB):
                nxt = None
                if j + 1 < NB:
                    if row_sc is not None:
                        row_sc.wait()
                        row_sc = None
                    nxt = pltpu.async_copy(
                        tables_h.at[c].at[nbr_b.at[j + 1]],
                        rows[(j + 1) % 2], sems[(j + 1) % 2])
                pend.wait()
                pend = nxt
                prev = row_sc
                row_sc = pltpu.async_copy(
                    rows[j % 2], a_nbr.at[self_b.at[j]], sem_r, add=True)
                if prev is not None:
                    prev.wait()
                small_sc.append(pltpu.async_copy(
                    dist_b.at[pl.ds(j * _CH, _CH), :],
                    a_ds.at[key_b.at[j]], sem_s, add=True))
                small_sc.append(pltpu.async_copy(
                    ones_v, a_ct.at[key_b.at[j]], sem_s, add=True))
            row_sc.wait()
            for d in small_sc:
                d.wait()
            return carry

        lax.fori_loop(0, nblocks, body, 0)
        plsc.subcore_barrier()
        pltpu.sync_copy(a_nbr.at[pl.ds(r0, RT), :], o_nbr.at[c, pl.ds(r0, RT), :])
        pltpu.sync_copy(a_ds.at[pl.ds(r0, RT), :], o_ds.at[c, pl.ds(r0, RT), :])
        pltpu.sync_copy(a_ct.at[pl.ds(r0, RT), :], o_ct.at[c, pl.ds(r0, RT), :])

    return sck(tables, keys, distp, zeros_h2, zeros_de, ones_src)


def _gram128(Rm, DE):
    BLK = 4096
    M = -(-Rm.shape[0] // BLK) * BLK
    if M != Rm.shape[0]:
        Rm = jnp.pad(Rm, ((0, M - Rm.shape[0]), (0, 0)))
    nblk = 128 // DE

    def gk(r_ref, o_ref):
        @pl.when(pl.program_id(0) == 0)
        def _init():
            o_ref[...] = jnp.zeros_like(o_ref)

        x = r_ref[...]
        o_ref[...] += lax.dot_general(
            x, x, (((0,), (0,)), ((), ())), preferred_element_type=jnp.float32)

    g128 = pl.pallas_call(
        gk,
        grid=(M // BLK,),
        in_specs=[pl.BlockSpec((BLK, 128), lambda i: (i, 0))],
        out_specs=pl.BlockSpec((128, 128), lambda i: (0, 0)),
        out_shape=jax.ShapeDtypeStruct((128, 128), jnp.float32),
    )(Rm)
    return sum(g128[DE * i:DE * (i + 1), DE * i:DE * (i + 1)]
               for i in range(nblk))


def _fused_dense(atom, nbrp, dsp, ctp, gdd_in,
                 w1a, w1b, b1, g1, be1, w2, b2, g2, be2, N, E):
    D = atom.shape[1]
    DH = D // 2
    DE = w1b.shape[1]
    Ef = float(E)

    def bk(atom_r, nbr_r, ds_r, ct_r, gdd_r,
           w1a0_r, w1a1_r, w1a_r, w1b_r, b1_r, g1_r, be1_r, w2_r, b2_r,
           g2_r, be2_r, o_r):
        nbr0 = nbr_r[0, :N, :]
        nbr1 = nbr_r[1, :N, :]
        dst = ds_r[0, :N, :]
        sv = ds_r[1, :N, :]
        cnt = ct_r[0, :N, 0:1]
        cb = ct_r[1, :N, 0:1]
        atom_v = atom_r[...]
        w1a0_v = w1a0_r[...]
        w1a1_v = w1a1_r[...]
        w1b_v = w1b_r[...]
        b1_v = b1_r[...]
        cc = jnp.maximum(cnt, 1.0)
        fea_pre = (
            lax.dot_general(nbr0 / cc, w1a0_v, (((1,), (1,)), ((), ())),
                            preferred_element_type=jnp.float32)
            + lax.dot_general(nbr1 / cc, w1a1_v, (((1,), (1,)), ((), ())),
                              preferred_element_type=jnp.float32)
            + lax.dot_general(dst / cc, w1b_v, (((1,), (1,)), ((), ())),
                              preferred_element_type=jnp.float32)
            + b1_v)
        g_nbr0 = jnp.sum(nbr0, axis=0, keepdims=True)
        g_nbr1 = jnp.sum(nbr1, axis=0, keepdims=True)
        g_dist = jnp.sum(dst, axis=0, keepdims=True)
        m1 = (
            lax.dot_general(g_nbr0, w1a0_v, (((1,), (1,)), ((), ())),
                            preferred_element_type=jnp.float32)
            + lax.dot_general(g_nbr1, w1a1_v, (((1,), (1,)), ((), ())),
                              preferred_element_type=jnp.float32)
            + lax.dot_general(g_dist, w1b_v, (((1,), (1,)), ((), ())),
                              preferred_element_type=jnp.float32)
        ) / Ef + b1_v
        gaa = lax.dot_general(atom_v * cb, atom_v, (((0,), (0,)), ((), ())),
                              preferred_element_type=jnp.float32)
        gad = lax.dot_general(atom_v, sv, (((0,), (0,)), ((), ())),
                              preferred_element_type=jnp.float32)
        gdd = gdd_r[...]
        w1a_v = w1a_r[...]
        t1 = lax.dot_general(w1a_v, gaa, (((1,), (0,)), ((), ())),
                             preferred_element_type=jnp.float32)
        t2 = lax.dot_general(w1a_v, gad, (((1,), (0,)), ((), ())),
                             preferred_element_type=jnp.float32)
        t3 = lax.dot_general(w1b_v, gdd, (((1,), (0,)), ((), ())),
                             preferred_element_type=jnp.float32)
        wgw = (jnp.sum(t1 * w1a_v, axis=1) + 2.0 * jnp.sum(t2 * w1b_v, axis=1)
               + jnp.sum(t3 * w1b_v, axis=1))
        m1f = m1[0, :]
        v1 = wgw / Ef + 2.0 * b1_v * m1f - b1_v * b1_v - m1f * m1f
        s1 = g1_r[...] / jnp.sqrt(v1 + 1e-5)
        mask = (cnt > 0.0).astype(jnp.float32)
        fea_summed = ((fea_pre - m1f) * s1 + be1_r[...]) * mask
        z = atom_v * mask
        h = lax.dot_general(z, w2_r[...], (((1,), (1,)), ((), ())),
                            preferred_element_type=jnp.float32) + b2_r[...]
        m2 = jnp.mean(h, axis=0)
        d2 = h - m2
        v2 = jnp.mean(d2 * d2, axis=0)
        xbn = d2 / jnp.sqrt(v2 + 1e-5) * g2_r[...] + be2_r[...] + fea_summed
        o_r[...] = jnp.maximum(xbn, 0.0) + jnp.log1p(jnp.exp(-jnp.abs(xbn)))

    return pl.pallas_call(
        bk,
        out_shape=jax.ShapeDtypeStruct((N, D), jnp.float32),
        compiler_params=pltpu.CompilerParams(
            vmem_limit_bytes=100 * 1024 * 1024),
    )(atom, nbrp, dsp, ctp, gdd_in,
      w1a[:, :DH], w1a[:, DH:], w1a, w1b, b1, g1, be1, w2, b2, g2, be2)


def kernel(atom_fea, nbr_dist_fea, nbr_adj_value, nbr_bond_type, self_fea_idx,
           nbr_fea_idx, ads_atom_idx, W1, b1, g1, be1, W2, b2, g2, be2):
    N, D = atom_fea.shape
    E, DE = nbr_dist_fea.shape
    DH = D // 2
    EW = -(-E // (_NS * _CH * 8)) * _CH * 8
    EP = EW * _NS
    RT = -(-(N + 1) // (_NS * 8)) * 8
    NP = RT * _NS

    f32 = jnp.float32
    selfp = jnp.pad(self_fea_idx.astype(jnp.int32), (0, EP - E),
                    constant_values=N)
    nbrp = jnp.pad(nbr_fea_idx.astype(jnp.int32), (0, EP - E),
                   constant_values=N)
    keys = jnp.stack([selfp, nbrp]).reshape(2, EP // _CH, _CH)
    distp = jnp.pad(nbr_dist_fea.astype(f32), ((0, EP - E), (0, 0)))
    table = jnp.pad(atom_fea.astype(f32), ((0, NP - N), (0, 0)))
    tables = jnp.stack([table[:, :DH], table[:, DH:]])
    zeros_h2 = jnp.zeros((RT, DH), f32)
    zeros_de = jnp.zeros((RT, DE), f32)
    ones_src = jnp.ones((_CH, DE), f32)

    nbr_part, ds_part, ct_part = _sc_segment_sums(
        tables, keys, distp, zeros_h2, zeros_de, ones_src, NP, RT, EW)

    gdd = _gram128(distp.reshape(EP * DE // 128, 128), DE)

    out = _fused_dense(
        atom_fea.astype(f32), nbr_part, ds_part, ct_part,
        gdd, W1[:, :D], W1[:, D:], b1, g1, be1, W2, b2, g2, be2, N, E)
    return out


# speedup vs baseline: 1.1956x; 1.0438x over previous
import functools

import jax
import jax.numpy as jnp
from jax import lax
from jax.experimental import pallas as pl
from jax.experimental.pallas import tpu as pltpu
from jax.experimental.pallas import tpu_sc as plsc

_NC = 2
_NS = 16
_CH = 128


def _sc_segment_sums(tables, keys, distp, zeros_h2, zeros_de, ones_src,
                     NP, RT, EW):
    DH = tables.shape[2]
    DE = distp.shape[1]
    NB = 16
    nblocks = EW // (_CH * NB)
    f32 = jnp.float32
    mesh = plsc.VectorSubcoreMesh(core_axis_name="c", subcore_axis_name="s")
    out_type = [
        jax.ShapeDtypeStruct((_NC, NP, DH), f32),
        jax.ShapeDtypeStruct((_NC, NP, DE), f32),
        jax.ShapeDtypeStruct((_NC, NP, DE), f32),
    ]
    scratch_types = [
        pltpu.VMEM((NB, _CH), jnp.int32),
        pltpu.VMEM((NB, _CH), jnp.int32),
        pltpu.VMEM((NB, _CH), jnp.int32),
        pltpu.VMEM((NB * _CH, DE), f32),
        pltpu.VMEM((_CH, DH), f32),
        pltpu.VMEM((_CH, DH), f32),
        pltpu.VMEM((_CH, DE), f32),
        pltpu.VMEM_SHARED((NP, DH), f32),
        pltpu.VMEM_SHARED((NP, DE), f32),
        pltpu.VMEM_SHARED((NP, DE), f32),
        pltpu.SemaphoreType.DMA,
        pltpu.SemaphoreType.DMA,
        pltpu.SemaphoreType.DMA,
        pltpu.SemaphoreType.DMA,
    ]

    @functools.partial(pl.kernel, mesh=mesh, out_type=out_type,
                       scratch_types=scratch_types,
                       compiler_params=pltpu.CompilerParams(
                           use_tc_tiling_on_sc=False))
    def sck(tables_h, keys_h, distp_h, zeros_h2_h, zeros_de_h, ones_h,
            o_nbr, o_ds, o_ct,
            self_b, nbr_b, key_b, dist_b, rows0, rows1, ones_v,
            a_nbr, a_ds, a_ct, sem0, sem1, sem_r, sem_s):
        c = lax.axis_index("c")
        s = lax.axis_index("s")
        r0 = s * RT
        pltpu.sync_copy(zeros_h2_h, a_nbr.at[pl.ds(r0, RT), :])
        pltpu.sync_copy(zeros_de_h, a_ds.at[pl.ds(r0, RT), :])
        pltpu.sync_copy(zeros_de_h, a_ct.at[pl.ds(r0, RT), :])
        pltpu.sync_copy(ones_h, ones_v)
        plsc.subcore_barrier()

        chunk0 = s * (EW // _CH)
        rows = (rows0, rows1)
        sems = (sem0, sem1)

        def body(b, carry):
            crow = chunk0 + b * NB
            eoff = crow * _CH
            pltpu.sync_copy(keys_h.at[0, pl.ds(crow, NB), :], self_b)
            pltpu.sync_copy(keys_h.at[1, pl.ds(crow, NB), :], nbr_b)
            pltpu.sync_copy(keys_h.at[c, pl.ds(crow, NB), :], key_b)
            pltpu.sync_copy(distp_h.at[pl.ds(eoff, NB * _CH), :], dist_b)
            pend = pltpu.async_copy(
                tables_h.at[c].at[nbr_b.at[0]], rows[0], sems[0])
            row_sc = None
            small_sc = []
            for j in range(NB):
                nxt = None
                if j + 1 < NB:
                    if row_sc is not None:
                        row_sc.wait()
                        row_sc = None
                    nxt = pltpu.async_copy(
                        tables_h.at[c].at[nbr_b.at[j + 1]],
                        rows[(j + 1) % 2], sems[(j + 1) % 2])
                pend.wait()
                pend = nxt
                prev = row_sc
                row_sc = pltpu.async_copy(
                    rows[j % 2], a_nbr.at[self_b.at[j]], sem_r, add=True)
                if prev is not None:
                    prev.wait()
                small_sc.append(pltpu.async_copy(
                    dist_b.at[pl.ds(j * _CH, _CH), :],
                    a_ds.at[key_b.at[j]], sem_s, add=True))
                small_sc.append(pltpu.async_copy(
                    ones_v, a_ct.at[key_b.at[j]], sem_s, add=True))
            row_sc.wait()
            for d in small_sc:
                d.wait()
            return carry

        lax.fori_loop(0, nblocks, body, 0)
        plsc.subcore_barrier()
        pltpu.sync_copy(a_nbr.at[pl.ds(r0, RT), :], o_nbr.at[c, pl.ds(r0, RT), :])
        pltpu.sync_copy(a_ds.at[pl.ds(r0, RT), :], o_ds.at[c, pl.ds(r0, RT), :])
        pltpu.sync_copy(a_ct.at[pl.ds(r0, RT), :], o_ct.at[c, pl.ds(r0, RT), :])

    return sck(tables, keys, distp, zeros_h2, zeros_de, ones_src)


def _gram128(Rm, DE):
    BLK = 4096
    M = -(-Rm.shape[0] // BLK) * BLK
    if M != Rm.shape[0]:
        Rm = jnp.pad(Rm, ((0, M - Rm.shape[0]), (0, 0)))
    nblk = 128 // DE

    def gk(r_ref, o_ref):
        @pl.when(pl.program_id(0) == 0)
        def _init():
            o_ref[...] = jnp.zeros_like(o_ref)

        x = r_ref[...]
        o_ref[...] += lax.dot_general(
            x, x, (((0,), (0,)), ((), ())), preferred_element_type=jnp.float32)

    g128 = pl.pallas_call(
        gk,
        grid=(M // BLK,),
        in_specs=[pl.BlockSpec((BLK, 128), lambda i: (i, 0))],
        out_specs=pl.BlockSpec((128, 128), lambda i: (0, 0)),
        out_shape=jax.ShapeDtypeStruct((128, 128), jnp.float32),
    )(Rm)
    return sum(g128[DE * i:DE * (i + 1), DE * i:DE * (i + 1)]
               for i in range(nblk))


def _fused_dense(atom, nbrp, dsp, ctp, gdd_in,
                 w1a, w1b, b1, g1, be1, w2, b2, g2, be2, N, E):
    D = atom.shape[1]
    DH = D // 2
    DE = w1b.shape[1]
    Ef = float(E)

    def bk(atom_r, nbr_r, ds_r, ct_r, gdd_r,
           w1a0_r, w1a1_r, w1a_r, w1b_r, b1_r, g1_r, be1_r, w2_r, b2_r,
           g2_r, be2_r, o_r):
        nbr0 = nbr_r[0, :N, :]
        nbr1 = nbr_r[1, :N, :]
        dst = ds_r[0, :N, :]
        sv = ds_r[1, :N, :]
        cnt = ct_r[0, :N, 0:1]
        cb = ct_r[1, :N, 0:1]
        atom_v = atom_r[...]
        w1a0_v = w1a0_r[...]
        w1a1_v = w1a1_r[...]
        w1b_v = w1b_r[...]
        b1_v = b1_r[...]
        cc = jnp.maximum(cnt, 1.0)
        fea_pre = (
            lax.dot_general(nbr0 / cc, w1a0_v, (((1,), (1,)), ((), ())),
                            preferred_element_type=jnp.float32)
            + lax.dot_general(nbr1 / cc, w1a1_v, (((1,), (1,)), ((), ())),
                              preferred_element_type=jnp.float32)
            + lax.dot_general(dst / cc, w1b_v, (((1,), (1,)), ((), ())),
                              preferred_element_type=jnp.float32)
            + b1_v)
        g_nbr0 = jnp.sum(nbr0, axis=0, keepdims=True)
        g_nbr1 = jnp.sum(nbr1, axis=0, keepdims=True)
        g_dist = jnp.sum(dst, axis=0, keepdims=True)
        m1 = (
            lax.dot_general(g_nbr0, w1a0_v, (((1,), (1,)), ((), ())),
                            preferred_element_type=jnp.float32)
            + lax.dot_general(g_nbr1, w1a1_v, (((1,), (1,)), ((), ())),
                              preferred_element_type=jnp.float32)
            + lax.dot_general(g_dist, w1b_v, (((1,), (1,)), ((), ())),
                              preferred_element_type=jnp.float32)
        ) / Ef + b1_v
        gaa = lax.dot_general(atom_v * cb, atom_v, (((0,), (0,)), ((), ())),
                              preferred_element_type=jnp.float32)
        gad = lax.dot_general(atom_v, sv, (((0,), (0,)), ((), ())),
                              preferred_element_type=jnp.float32)
        gdd = gdd_r[...]
        w1a_v = w1a_r[...]
        t1 = lax.dot_general(w1a_v, gaa, (((1,), (0,)), ((), ())),
                             preferred_element_type=jnp.float32)
        t2 = lax.dot_general(w1a_v, gad, (((1,), (0,)), ((), ())),
                             preferred_element_type=jnp.float32)
        t3 = lax.dot_general(w1b_v, gdd, (((1,), (0,)), ((), ())),
                             preferred_element_type=jnp.float32)
        wgw = (jnp.sum(t1 * w1a_v, axis=1) + 2.0 * jnp.sum(t2 * w1b_v, axis=1)
               + jnp.sum(t3 * w1b_v, axis=1))
        m1f = m1[0, :]
        v1 = wgw / Ef + 2.0 * b1_v * m1f - b1_v * b1_v - m1f * m1f
        s1 = g1_r[...] / jnp.sqrt(v1 + 1e-5)
        mask = (cnt > 0.0).astype(jnp.float32)
        fea_summed = ((fea_pre - m1f) * s1 + be1_r[...]) * mask
        z = atom_v * mask
        h = lax.dot_general(z, w2_r[...], (((1,), (1,)), ((), ())),
                            preferred_element_type=jnp.float32) + b2_r[...]
        m2 = jnp.mean(h, axis=0)
        d2 = h - m2
        v2 = jnp.mean(d2 * d2, axis=0)
        xbn = d2 / jnp.sqrt(v2 + 1e-5) * g2_r[...] + be2_r[...] + fea_summed
        o_r[...] = jnp.maximum(xbn, 0.0) + jnp.log1p(jnp.exp(-jnp.abs(xbn)))

    return pl.pallas_call(
        bk,
        out_shape=jax.ShapeDtypeStruct((N, D), jnp.float32),
        compiler_params=pltpu.CompilerParams(
            vmem_limit_bytes=100 * 1024 * 1024),
    )(atom, nbrp, dsp, ctp, gdd_in,
      w1a[:, :DH], w1a[:, DH:], w1a, w1b, b1, g1, be1, w2, b2, g2, be2)


def kernel(atom_fea, nbr_dist_fea, nbr_adj_value, nbr_bond_type, self_fea_idx,
           nbr_fea_idx, ads_atom_idx, W1, b1, g1, be1, W2, b2, g2, be2):
    N, D = atom_fea.shape
    E, DE = nbr_dist_fea.shape
    DH = D // 2
    EW = -(-E // (_NS * _CH * 16)) * _CH * 16
    EP = EW * _NS
    RT = -(-(N + 1) // (_NS * 8)) * 8
    NP = RT * _NS

    f32 = jnp.float32
    selfp = jnp.pad(self_fea_idx.astype(jnp.int32), (0, EP - E),
                    constant_values=N)
    nbrp = jnp.pad(nbr_fea_idx.astype(jnp.int32), (0, EP - E),
                   constant_values=N)
    keys = jnp.stack([selfp, nbrp]).reshape(2, EP // _CH, _CH)
    distp = jnp.pad(nbr_dist_fea.astype(f32), ((0, EP - E), (0, 0)))
    table = jnp.pad(atom_fea.astype(f32), ((0, NP - N), (0, 0)))
    tables = jnp.stack([table[:, :DH], table[:, DH:]])
    zeros_h2 = jnp.zeros((RT, DH), f32)
    zeros_de = jnp.zeros((RT, DE), f32)
    ones_src = jnp.ones((_CH, DE), f32)

    nbr_part, ds_part, ct_part = _sc_segment_sums(
        tables, keys, distp, zeros_h2, zeros_de, ones_src, NP, RT, EW)

    gdd = _gram128(distp.reshape(EP * DE // 128, 128), DE)

    out = _fused_dense(
        atom_fea.astype(f32), nbr_part, ds_part, ct_part,
        gdd, W1[:, :D], W1[:, D:], b1, g1, be1, W2, b2, g2, be2, N, E)
    return out
